# baseline (speedup 1.0000x reference)
import functools

import jax
import jax.numpy as jnp
from jax import lax
from jax.experimental import pallas as pl
from jax.experimental.pallas import tpu as pltpu
from jax.experimental.pallas import tpu_sc as plsc

B = 4096
V = 100000
D = 16
L = 50
NC = 2
NS = 16
NW = NC * NS
BPT = B // NW
INV_L = 1.0 / L


def _tree_sum(vals):
    while len(vals) > 1:
        nxt = [vals[i] + vals[i + 1] for i in range(0, len(vals) - 1, 2)]
        if len(vals) % 2:
            nxt.append(vals[-1])
        vals = nxt
    return vals[0]


def kernel(input_0, input_1, input_2, input_3,
           W_0_1, W_0_2, W_0_3,
           W_1_0, W_1_2, W_1_3,
           W_2_0, W_2_1, W_2_3,
           W_3_0, W_3_1, W_3_2):
    mesh = plsc.VectorSubcoreMesh(core_axis_name="c", subcore_axis_name="s")

    @functools.partial(
        pl.kernel,
        mesh=mesh,
        compiler_params=pltpu.CompilerParams(
            needs_layout_passes=False, use_tc_tiling_on_sc=False),
        out_type=jax.ShapeDtypeStruct((B,), jnp.float32),
        scratch_types=[
            pltpu.VMEM((BPT,), jnp.int32),
            pltpu.VMEM((BPT,), jnp.int32),
            pltpu.VMEM((BPT,), jnp.int32),
            pltpu.VMEM((BPT, L), jnp.int32),
        ] + [pltpu.VMEM((D, BPT), jnp.int32) for _ in range(3)]
          + [pltpu.VMEM((D, 136), jnp.float32) for _ in range(9)]
          + [pltpu.VMEM((L, D), jnp.float32) for _ in range(6)]
          + [
            pltpu.VMEM((BPT,), jnp.float32),
            pltpu.SemaphoreType.DMA,
            pltpu.SemaphoreType.DMA,
            pltpu.SemaphoreType.DMA,
        ],
    )
    def k(i0, i1, i2, i3,
          w01, w02, w03, w10, w12, w13, w20, w21, w23, w30, w31, w32,
          out,
          idx0_v, idx1_v, idx2_v, idx3_v,
          im0, im1, im2,
          r01, r02, r03, r10, r12, r13, r20, r21, r23,
          s0a, s1a, s2a, s0b, s1b, s2b,
          out_v,
          sem_sc, sem_a, sem_b):
        wid = lax.axis_index("s") * NC + lax.axis_index("c")
        base = wid * BPT

        pltpu.sync_copy(i0.at[pl.ds(base, BPT)], idx0_v)
        pltpu.sync_copy(i1.at[pl.ds(base, BPT)], idx1_v)
        pltpu.sync_copy(i2.at[pl.ds(base, BPT)], idx2_v)
        pltpu.sync_copy(i3.at[pl.ds(base, BPT)], idx3_v)

        for im, idxv in ((im0, idx0_v), (im1, idx1_v), (im2, idx2_v)):
            for f in range(D):
                for j in range(BPT // D):
                    im[f, pl.ds(j * D, D)] = (
                        idxv[pl.ds(j * D, D)] + f * V)

        field_tabs = ((im0, (w01, r01), (w02, r02), (w03, r03)),
                      (im1, (w10, r10), (w12, r12), (w13, r13)),
                      (im2, (w20, r20), (w21, r21), (w23, r23)))
        sc_copies = []
        for im, *tabs in field_tabs:
            for w, rbuf in tabs:
                for f in range(D):
                    sc_copies.append(pltpu.make_async_copy(
                        w.at[im.at[f]], rbuf.at[f, pl.ds(0, BPT)], sem_sc))
        for c in sc_copies:
            c.start()
        for c in sc_copies:
            c.wait()

        ring0 = (s0a, s1a, s2a)
        ring1 = (s0b, s1b, s2b)

        def issue(b, bufs, sem):
            idxrow = idx3_v.at[b]
            pltpu.make_async_copy(w30.at[idxrow], bufs[0], sem).start()
            pltpu.make_async_copy(w31.at[idxrow], bufs[1], sem).start()
            pltpu.make_async_copy(w32.at[idxrow], bufs[2], sem).start()

        def wait3(bufs, sem):
            for buf in bufs:
                pltpu.make_async_copy(w30.at[idx3_v.at[0]], buf, sem).wait()

        lanes = lax.iota(jnp.int32, D)
        perms = [lanes ^ sh for sh in (8, 4, 2, 1)]

        def colload(rbuf, b):
            bvec = jnp.zeros((D,), jnp.int32) + b
            return plsc.load_gather(rbuf, [lanes, bvec])

        gdn = lax.GatherDimensionNumbers(
            offset_dims=(), collapsed_slice_dims=(0,), start_index_map=(0,))

        def allsum(v):
            for perm in perms:
                v = v + lax.gather(
                    v, perm[:, None], dimension_numbers=gdn, slice_sizes=(1,),
                    mode=lax.GatherScatterMode.PROMISE_IN_BOUNDS)
            return v

        def compute(b, bufs):
            m0 = _tree_sum([bufs[0][l] for l in range(L)])
            m1 = _tree_sum([bufs[1][l] for l in range(L)])
            m2 = _tree_sum([bufs[2][l] for l in range(L)])
            p = (colload(r01, b) * colload(r10, b)
                 + colload(r02, b) * colload(r20, b)
                 + colload(r12, b) * colload(r21, b)
                 + (colload(r03, b) * m0 + colload(r13, b) * m1
                    + colload(r23, b) * m2) * INV_L)
            return allsum(p)

        issue(0, ring0, sem_a)

        def body(t, acc):
            b0 = 2 * t
            lane0 = b0 % D
            issue(b0 + 1, ring1, sem_b)
            wait3(ring0, sem_a)
            s0 = compute(b0, ring0)
            issue(jnp.minimum(b0 + 2, BPT - 1), ring0, sem_a)
            wait3(ring1, sem_b)
            s1 = compute(b0 + 1, ring1)
            acc = jnp.where(lanes == lane0, s0, acc)
            acc = jnp.where(lanes == lane0 + 1, s1, acc)

            @pl.when(t % (D // 2) == (D // 2) - 1)
            def _():
                out_v[pl.ds((t // (D // 2)) * D, D)] = acc

            return acc

        lax.fori_loop(0, BPT // 2, body, jnp.zeros((D,), jnp.float32))
        wait3(ring0, sem_a)

        pltpu.sync_copy(out_v, out.at[pl.ds(base, BPT)])

    out_flat = k(input_0.reshape(B), input_1.reshape(B), input_2.reshape(B),
                 input_3,
                 *[w.T.reshape(V * D) for w in (W_0_1, W_0_2, W_0_3,
                                                W_1_0, W_1_2, W_1_3,
                                                W_2_0, W_2_1, W_2_3)],
                 W_3_0, W_3_1, W_3_2)
    return out_flat.reshape(B, 1, 1)
